# baseline (speedup 1.0000x reference)
import functools

import jax
import jax.numpy as jnp
from jax import lax
from jax.experimental import pallas as pl
from jax.experimental.pallas import tpu as pltpu
from jax.experimental.pallas import tpu_sc as plsc

B = 4096
BLK = 1024
BLKG = 256
P_MAX = 10240
NBLK_MAX = P_MAX // BLKG
META_N = 64
NW = 16
TPW = B // NW
N_COINS = 250
COIN_DIM = 32
REG_DIM = 16
N_ACC = 4
N_TEMP = 40
EH = 256
EO = 128
NE = 8
FEAT_DIM = 256
PART_NAMES = ("price", "volume", "orderflow", "derived")

_SQRT2 = 1.4142135623730951
_RSQRT_EO = 1.0 / (EO ** 0.5)


def _gelu(x):
    return 0.5 * x * (1.0 + lax.erf(x / _SQRT2))


def _ln(x, g, b, eps=1e-5):
    m = jnp.mean(x, axis=-1, keepdims=True)
    xc = x - m
    v = jnp.mean(xc * xc, axis=-1, keepdims=True)
    return xc * lax.rsqrt(v + eps) * g + b


def _dot(x, w):
    return jnp.dot(x, w, preferred_element_type=jnp.float32)


def _flatten_params(p):
    f32 = jnp.float32
    w = {}
    w["coin_emb"] = jnp.zeros((256, COIN_DIM), f32).at[:N_COINS].set(
        p["coin_emb"])
    w["regime_emb"] = jnp.zeros((8, REG_DIM), f32).at[:4].set(p["regime_emb"])
    w["temp1_w"] = p["temp1"]["w"]
    w["temp1_b"] = p["temp1"]["b"][None]
    w["temp2_w"] = p["temp2"]["w"]
    w["temp2_b"] = p["temp2"]["b"][None]
    w["temp_lng"] = p["temp_lng"][None]
    w["temp_lnb"] = p["temp_lnb"][None]
    for i, name in enumerate(PART_NAMES):
        ep = p["feat_experts"][name]
        w[f"fe{i}_w1"] = ep["w1"]
        w[f"fe{i}_b1"] = ep["b1"][None]
        w[f"fe{i}_w2"] = ep["w2"]
        w[f"fe{i}_b2"] = ep["b2"][None]
        w[f"fe{i}_w3"] = ep["w3"]
        w[f"fe{i}_b3"] = ep["b3"][None]
        w[f"fe{i}_wr"] = ep["wr"]
        w[f"fe{i}_br"] = ep["br"][None]
        w[f"fe{i}_lng"] = ep["lng"][None]
        w[f"fe{i}_lnb"] = ep["lnb"][None]
        w[f"gk{i}_w"] = p["gate_keys"][name]["w"]
        w[f"gk{i}_b"] = p["gate_keys"][name]["b"][None]
    cw = p["context"]["w"]
    w["ctx_wa"] = cw[0:N_ACC]
    w["ctx_wc"] = cw[N_ACC:N_ACC + COIN_DIM]
    w["ctx_wr"] = cw[N_ACC + COIN_DIM:N_ACC + COIN_DIM + REG_DIM]
    w["ctx_wt"] = cw[N_ACC + COIN_DIM + REG_DIM:]
    w["ctx_b"] = p["context"]["b"][None]
    qw = p["gate_q"]["w"]
    w["gq_cat"] = qw[:4 * EO]
    w["gq_ctx"] = qw[4 * EO:]
    w["gq_b"] = p["gate_q"]["b"][None]
    rw = p["router1"]["w"]
    w["r1_g"] = rw[:EO]
    w["r1_r"] = rw[EO:]
    w["r1_b"] = p["router1"]["b"][None]
    w["r2_w"] = p["router2"]["w"]
    w["r2_b"] = p["router2"]["b"][None]
    for e in range(NE):
        ep = p["moe_experts"][e]
        w[f"moe{e}_w1"] = ep["w1"]
        w[f"moe{e}_b1"] = ep["b1"][None]
        w[f"moe{e}_w2"] = ep["w2"]
        w[f"moe{e}_b2"] = ep["b2"][None]
        w[f"moe{e}_w3"] = ep["w3"]
        w[f"moe{e}_b3"] = ep["b3"][None]
        w[f"moe{e}_lng"] = ep["lng"][None]
        w[f"moe{e}_lnb"] = ep["lnb"][None]
    fw = p["fus1"]["w"]
    w["f1_m"] = fw[:EO]
    w["f1_c"] = fw[EO:]
    w["f1_b"] = p["fus1"]["b"][None]
    w["f_ln1g"] = p["fus_ln1g"][None]
    w["f_ln1b"] = p["fus_ln1b"][None]
    w["f2_w"] = p["fus2"]["w"]
    w["f2_b"] = p["fus2"]["b"][None]
    w["f_ln2g"] = p["fus_ln2g"][None]
    w["f_ln2b"] = p["fus_ln2b"][None]
    for g in range(4):
        hp = p["heads"][g]
        for hname in ("lab", "mae", "mfe"):
            for lyr in ("1", "2"):
                w[f"hd_{hname}{lyr}_{g}_w"] = hp[hname + lyr]["w"]
                w[f"hd_{hname}{lyr}_{g}_b"] = hp[hname + lyr]["b"][None]
    for nm in ("conf1", "conf2", "lev1", "lev2"):
        w[nm + "_w"] = p[nm]["w"]
        w[nm + "_b"] = p[nm]["b"][None]
    return w


def _body(names, *refs):
    feats_ref, coin_ref, reg_ref, acct_ref, temp_ref = refs[:5]
    out_ref = refs[-1]
    w = {n: r for n, r in zip(names, refs[5:-1])}

    feats = feats_ref[...]
    coin_id = coin_ref[...]
    regime_id = reg_ref[...]
    acct = acct_ref[...]
    temporal = temp_ref[...]

    iota_c = lax.broadcasted_iota(jnp.int32, (BLK, 256), 1)
    oh_c = (iota_c == coin_id).astype(jnp.float32)
    coin_emb = _dot(oh_c, w["coin_emb"][...])
    iota_r = lax.broadcasted_iota(jnp.int32, (BLK, 8), 1)
    oh_r = (iota_r == regime_id).astype(jnp.float32)
    regime_emb = _dot(oh_r, w["regime_emb"][...])

    t = _gelu(_dot(temporal, w["temp1_w"][...]) + w["temp1_b"][...])
    t = _dot(t, w["temp2_w"][...]) + w["temp2_b"][...]
    temporal_enc = _ln(t, w["temp_lng"][...], w["temp_lnb"][...])

    feat_outs = []
    for i in range(4):
        x = feats[:, i * 64:(i + 1) * 64]
        h = _gelu(_dot(x, w[f"fe{i}_w1"][...]) + w[f"fe{i}_b1"][...])
        h = _gelu(_dot(h, w[f"fe{i}_w2"][...]) + w[f"fe{i}_b2"][...])
        h = _dot(h, w[f"fe{i}_w3"][...]) + w[f"fe{i}_b3"][...]
        res = _dot(x, w[f"fe{i}_wr"][...]) + w[f"fe{i}_br"][...]
        feat_outs.append(_ln(h + res, w[f"fe{i}_lng"][...],
                             w[f"fe{i}_lnb"][...]))

    ctx = (_dot(acct, w["ctx_wa"][...]) + _dot(coin_emb, w["ctx_wc"][...])
           + _dot(regime_emb, w["ctx_wr"][...])
           + _dot(temporal_enc, w["ctx_wt"][...]) + w["ctx_b"][...])
    context_enc = _gelu(ctx)

    fcat = jnp.concatenate(feat_outs, axis=-1)
    q = (w["gq_b"][...] + _dot(context_enc, w["gq_ctx"][...])
         + _dot(fcat, w["gq_cat"][...]))
    scores = []
    for i in range(4):
        k = _dot(feat_outs[i], w[f"gk{i}_w"][...]) + w[f"gk{i}_b"][...]
        scores.append(jnp.sum(q * k, axis=-1, keepdims=True) * _RSQRT_EO)
    smax = jnp.maximum(jnp.maximum(scores[0], scores[1]),
                       jnp.maximum(scores[2], scores[3]))
    exps = [jnp.exp(s - smax) for s in scores]
    denom = exps[0] + exps[1] + exps[2] + exps[3]
    gated = jnp.zeros((BLK, EO), jnp.float32)
    for i in range(4):
        gated = gated + (exps[i] / denom) * feat_outs[i]

    rh = _gelu(_dot(gated, w["r1_g"][...]) + _dot(regime_emb, w["r1_r"][...])
               + w["r1_b"][...])
    logits = _dot(rh, w["r2_w"][...]) + w["r2_b"][...]
    iota8 = lax.broadcasted_iota(jnp.int32, (BLK, NE), 1)
    m1 = jnp.max(logits, axis=-1, keepdims=True)
    i1 = jnp.min(jnp.where(logits == m1, iota8, NE), axis=-1, keepdims=True)
    masked = jnp.where(iota8 == i1, -1e30, logits)
    m2 = jnp.max(masked, axis=-1, keepdims=True)
    i2 = jnp.min(jnp.where(masked == m2, iota8, NE), axis=-1, keepdims=True)
    e2 = jnp.exp(m2 - m1)
    w1c = 1.0 / (1.0 + e2)
    w2c = e2 * w1c
    coefs = (jnp.where(iota8 == i1, w1c, 0.0)
             + jnp.where(iota8 == i2, w2c, 0.0))

    moe = jnp.zeros((BLK, EO), jnp.float32)
    for e in range(NE):
        h = _gelu(_dot(gated, w[f"moe{e}_w1"][...]) + w[f"moe{e}_b1"][...])
        h = _gelu(_dot(h, w[f"moe{e}_w2"][...]) + w[f"moe{e}_b2"][...])
        h = _dot(h, w[f"moe{e}_w3"][...]) + w[f"moe{e}_b3"][...]
        eo = _ln(h + gated, w[f"moe{e}_lng"][...], w[f"moe{e}_lnb"][...])
        moe = moe + lax.slice_in_dim(coefs, e, e + 1, axis=1) * eo

    f = _gelu(_dot(moe, w["f1_m"][...]) + _dot(context_enc, w["f1_c"][...])
              + w["f1_b"][...])
    f = _ln(f, w["f_ln1g"][...], w["f_ln1b"][...])
    f = _gelu(_dot(f, w["f2_w"][...]) + w["f2_b"][...])
    f = _ln(f, w["f_ln2g"][...], w["f_ln2b"][...])

    pieces = []
    for hname in ("lab", "mae", "mfe"):
        for g in range(4):
            h1 = _gelu(_dot(f, w[f"hd_{hname}1_{g}_w"][...])
                       + w[f"hd_{hname}1_{g}_b"][...])
            pieces.append(_dot(h1, w[f"hd_{hname}2_{g}_w"][...])
                          + w[f"hd_{hname}2_{g}_b"][...])
    c = _gelu(_dot(f, w["conf1_w"][...]) + w["conf1_b"][...])
    pieces.append(jax.nn.sigmoid(_dot(c, w["conf2_w"][...])
                                 + w["conf2_b"][...]))
    lv = _gelu(_dot(f, w["lev1_w"][...]) + w["lev1_b"][...])
    pieces.append(jax.nn.sigmoid(_dot(lv, w["lev2_w"][...])
                                 + w["lev2_b"][...]))
    out_ref[...] = jnp.concatenate(pieces, axis=-1)


def _const_spec(arr):
    nd = arr.ndim
    return pl.BlockSpec(arr.shape, lambda i, _nd=nd: (0,) * _nd)


def _is_post(n):
    return (n.startswith(("f1_", "f2_", "f_ln", "hd_"))
            or n.startswith(("conf", "lev")))


def _is_pre(n):
    return not _is_post(n) and not n.startswith("moe")


def _pre_body(names, *refs):
    feats_ref, coin_ref, reg_ref, acct_ref, temp_ref = refs[:5]
    gated_ref, ctx_ref, eidx_ref, ew_ref = refs[-4:]
    w = {n: r for n, r in zip(names, refs[5:-4])}

    feats = feats_ref[...]
    coin_id = coin_ref[...]
    regime_id = reg_ref[...]
    acct = acct_ref[...]
    temporal = temp_ref[...]

    iota_c = lax.broadcasted_iota(jnp.int32, (BLK, 256), 1)
    oh_c = (iota_c == coin_id).astype(jnp.float32)
    coin_emb = _dot(oh_c, w["coin_emb"][...])
    iota_r = lax.broadcasted_iota(jnp.int32, (BLK, 8), 1)
    oh_r = (iota_r == regime_id).astype(jnp.float32)
    regime_emb = _dot(oh_r, w["regime_emb"][...])

    t = _gelu(_dot(temporal, w["temp1_w"][...]) + w["temp1_b"][...])
    t = _dot(t, w["temp2_w"][...]) + w["temp2_b"][...]
    temporal_enc = _ln(t, w["temp_lng"][...], w["temp_lnb"][...])

    feat_outs = []
    for i in range(4):
        x = feats[:, i * 64:(i + 1) * 64]
        h = _gelu(_dot(x, w[f"fe{i}_w1"][...]) + w[f"fe{i}_b1"][...])
        h = _gelu(_dot(h, w[f"fe{i}_w2"][...]) + w[f"fe{i}_b2"][...])
        h = _dot(h, w[f"fe{i}_w3"][...]) + w[f"fe{i}_b3"][...]
        res = _dot(x, w[f"fe{i}_wr"][...]) + w[f"fe{i}_br"][...]
        feat_outs.append(_ln(h + res, w[f"fe{i}_lng"][...],
                             w[f"fe{i}_lnb"][...]))

    ctx = (_dot(acct, w["ctx_wa"][...]) + _dot(coin_emb, w["ctx_wc"][...])
           + _dot(regime_emb, w["ctx_wr"][...])
           + _dot(temporal_enc, w["ctx_wt"][...]) + w["ctx_b"][...])
    context_enc = _gelu(ctx)

    fcat = jnp.concatenate(feat_outs, axis=-1)
    q = (w["gq_b"][...] + _dot(context_enc, w["gq_ctx"][...])
         + _dot(fcat, w["gq_cat"][...]))
    scores = []
    for i in range(4):
        k = _dot(feat_outs[i], w[f"gk{i}_w"][...]) + w[f"gk{i}_b"][...]
        scores.append(jnp.sum(q * k, axis=-1, keepdims=True) * _RSQRT_EO)
    smax = jnp.maximum(jnp.maximum(scores[0], scores[1]),
                       jnp.maximum(scores[2], scores[3]))
    exps = [jnp.exp(s - smax) for s in scores]
    denom = exps[0] + exps[1] + exps[2] + exps[3]
    gated = jnp.zeros((BLK, EO), jnp.float32)
    for i in range(4):
        gated = gated + (exps[i] / denom) * feat_outs[i]

    rh = _gelu(_dot(gated, w["r1_g"][...]) + _dot(regime_emb, w["r1_r"][...])
               + w["r1_b"][...])
    logits = _dot(rh, w["r2_w"][...]) + w["r2_b"][...]
    iota8 = lax.broadcasted_iota(jnp.int32, (BLK, NE), 1)
    m1 = jnp.max(logits, axis=-1, keepdims=True)
    i1 = jnp.min(jnp.where(logits == m1, iota8, NE), axis=-1, keepdims=True)
    masked = jnp.where(iota8 == i1, -1e30, logits)
    m2 = jnp.max(masked, axis=-1, keepdims=True)
    i2 = jnp.min(jnp.where(masked == m2, iota8, NE), axis=-1, keepdims=True)
    e2 = jnp.exp(m2 - m1)
    w1c = 1.0 / (1.0 + e2)
    w2c = e2 * w1c

    gated_ref[...] = gated
    ctx_ref[...] = context_enc
    eidx_ref[...] = jnp.where(iota8 == 0, i1, jnp.where(iota8 == 1, i2, 0))
    ew_ref[...] = jnp.where(iota8 == 0, w1c, jnp.where(iota8 == 1, w2c, 0.0))


def _post_body(names, *refs):
    m0_ref, m1_ref, w1_ref, w2_ref, ctx_ref = refs[:5]
    out_ref = refs[-1]
    w = {n: r for n, r in zip(names, refs[5:-1])}
    moe = w1_ref[...] * m0_ref[...] + w2_ref[...] * m1_ref[...]
    context_enc = ctx_ref[...]
    f = _gelu(_dot(moe, w["f1_m"][...]) + _dot(context_enc, w["f1_c"][...])
              + w["f1_b"][...])
    f = _ln(f, w["f_ln1g"][...], w["f_ln1b"][...])
    f = _gelu(_dot(f, w["f2_w"][...]) + w["f2_b"][...])
    f = _ln(f, w["f_ln2g"][...], w["f_ln2b"][...])
    pieces = []
    for hname in ("lab", "mae", "mfe"):
        for g in range(4):
            h1 = _gelu(_dot(f, w[f"hd_{hname}1_{g}_w"][...])
                       + w[f"hd_{hname}1_{g}_b"][...])
            pieces.append(_dot(h1, w[f"hd_{hname}2_{g}_w"][...])
                          + w[f"hd_{hname}2_{g}_b"][...])
    c = _gelu(_dot(f, w["conf1_w"][...]) + w["conf1_b"][...])
    pieces.append(jax.nn.sigmoid(_dot(c, w["conf2_w"][...])
                                 + w["conf2_b"][...]))
    lv = _gelu(_dot(f, w["lev1_w"][...]) + w["lev1_b"][...])
    pieces.append(jax.nn.sigmoid(_dot(lv, w["lev2_w"][...])
                                 + w["lev2_b"][...]))
    out_ref[...] = jnp.concatenate(pieces, axis=-1)


def _grouped_body(m_ref, x_ref, w1_ref, w2_ref, w3_ref, b1_ref, b2_ref,
                  b3_ref, lng_ref, lnb_ref, o_ref):
    i = pl.program_id(0)

    @pl.when(i < m_ref[48])
    def _():
        x = x_ref[...]
        h = _gelu(_dot(x, w1_ref[0]) + b1_ref[0])
        h = _gelu(_dot(h, w2_ref[0]) + b2_ref[0])
        h = _dot(h, w3_ref[0]) + b3_ref[0]
        o_ref[...] = _ln(h + x, lng_ref[0], lnb_ref[0])


def _run_grouped(gathered, meta, w, interpret=False):
    grid_spec = pltpu.PrefetchScalarGridSpec(
        num_scalar_prefetch=1,
        grid=(NBLK_MAX,),
        in_specs=[
            pl.BlockSpec((BLKG, EO), lambda i, m: (i, 0)),
            pl.BlockSpec((1, EO, EH), lambda i, m: (m[i], 0, 0)),
            pl.BlockSpec((1, EH, EH), lambda i, m: (m[i], 0, 0)),
            pl.BlockSpec((1, EH, EO), lambda i, m: (m[i], 0, 0)),
            pl.BlockSpec((1, 1, EH), lambda i, m: (m[i], 0, 0)),
            pl.BlockSpec((1, 1, EH), lambda i, m: (m[i], 0, 0)),
            pl.BlockSpec((1, 1, EO), lambda i, m: (m[i], 0, 0)),
            pl.BlockSpec((1, 1, EO), lambda i, m: (m[i], 0, 0)),
            pl.BlockSpec((1, 1, EO), lambda i, m: (m[i], 0, 0)),
        ],
        out_specs=pl.BlockSpec((BLKG, EO), lambda i, m: (i, 0)),
    )
    mw1 = jnp.stack([w[f"moe{e}_w1"] for e in range(NE)])
    mw2 = jnp.stack([w[f"moe{e}_w2"] for e in range(NE)])
    mw3 = jnp.stack([w[f"moe{e}_w3"] for e in range(NE)])
    mb1 = jnp.stack([w[f"moe{e}_b1"] for e in range(NE)])
    mb2 = jnp.stack([w[f"moe{e}_b2"] for e in range(NE)])
    mb3 = jnp.stack([w[f"moe{e}_b3"] for e in range(NE)])
    mlg = jnp.stack([w[f"moe{e}_lng"] for e in range(NE)])
    mlb = jnp.stack([w[f"moe{e}_lnb"] for e in range(NE)])
    return pl.pallas_call(
        _grouped_body,
        grid_spec=grid_spec,
        out_shape=jax.ShapeDtypeStruct((P_MAX, EO), jnp.float32),
        interpret=interpret,
    )(meta, gathered, mw1, mw2, mw3, mb1, mb2, mb3, mlg, mlb)


def _dispatch_emul(gated, e1, e2):
    eall = jnp.concatenate([e1, e2])
    toks = jnp.concatenate([jnp.arange(B), jnp.arange(B)])
    counts = jnp.sum((eall[:, None] == jnp.arange(NE)[None, :]), axis=0)
    padded = ((counts + BLKG - 1) // BLKG) * BLKG
    off = jnp.cumsum(padded) - padded
    oh = (eall[:, None] == jnp.arange(NE)[None, :]).astype(jnp.int32)
    rank = jnp.cumsum(oh, axis=0) - oh
    pos = off[eall] + jnp.take_along_axis(rank, eall[:, None], 1)[:, 0]
    gathered = jnp.zeros((P_MAX, EO), jnp.float32).at[pos].set(gated[toks])
    nblocks = jnp.sum(padded) // BLKG
    starts = jnp.arange(NBLK_MAX) * BLKG
    bexp = jnp.sum(jnp.where(
        (starts[:, None] >= off[None, :])
        & (starts[:, None] < (off + padded)[None, :]),
        jnp.arange(NE)[None, :], 0), axis=1)
    meta = jnp.zeros((META_N,), jnp.int32)
    meta = meta.at[:NBLK_MAX].set(bexp).at[48].set(nblocks)
    return gathered, pos[:B], pos[B:], meta


def _combine_emul(eo, pos0, pos1, w1col, w2col):
    return w1col[:, None] * eo[pos0] + w2col[:, None] * eo[pos1]


def _sc_hist(e1, e2):
    mesh = plsc.VectorSubcoreMesh(core_axis_name="c", subcore_axis_name="s",
                                  num_cores=1)
    nv = TPW // 16

    @functools.partial(
        pl.kernel, mesh=mesh,
        compiler_params=pltpu.CompilerParams(needs_layout_passes=False),
        out_type=jax.ShapeDtypeStruct((NW, 16), jnp.int32),
        scratch_types=[
            pltpu.VMEM((TPW,), jnp.int32),
            pltpu.VMEM((TPW,), jnp.int32),
            pltpu.VMEM((16,), jnp.int32),
        ],
    )
    def _histk(e1_hbm, e2_hbm, hist_hbm, e1_v, e2_v, hist_v):
        wid = lax.axis_index("s")
        base = wid * TPW
        lane = lax.iota(jnp.int32, 16)
        pltpu.sync_copy(e1_hbm.at[pl.ds(base, TPW)], e1_v)
        pltpu.sync_copy(e2_hbm.at[pl.ds(base, TPW)], e2_v)
        hist = jnp.zeros((16,), jnp.int32)
        for src in (e1_v, e2_v):
            for j in range(nv):
                v = src[pl.ds(j * 16, 16)]
                for e in range(NE):
                    c = jnp.sum((v == e).astype(jnp.int32))
                    hist = jnp.where(lane == e, hist + c, hist)
        hist_v[...] = hist
        pltpu.sync_copy(hist_v, hist_hbm.at[wid])

    return _histk(e1, e2)


def _sc_dispatch(gated, e1, e2):
    hist = _sc_hist(e1, e2)
    mesh = plsc.VectorSubcoreMesh(core_axis_name="c", subcore_axis_name="s",
                                  num_cores=1)
    nv = TPW // 16
    nr = TPW // 128

    @functools.partial(
        pl.kernel, mesh=mesh,
        compiler_params=pltpu.CompilerParams(needs_layout_passes=False),
        out_type=[
            jax.ShapeDtypeStruct((P_MAX, EO), jnp.float32),
            jax.ShapeDtypeStruct((B // 128, 128), jnp.int32),
            jax.ShapeDtypeStruct((B // 128, 128), jnp.int32),
            jax.ShapeDtypeStruct((META_N,), jnp.int32),
        ],
        scratch_types=[
            pltpu.VMEM((TPW,), jnp.int32),
            pltpu.VMEM((TPW,), jnp.int32),
            pltpu.VMEM((TPW, EO), jnp.float32),
            pltpu.VMEM((2, 128), jnp.int32),
            pltpu.VMEM((2, 128), jnp.int32),
            pltpu.VMEM((NW, 16), jnp.int32),
            pltpu.VMEM((META_N,), jnp.int32),
            pltpu.SemaphoreType.DMA,
            pltpu.SemaphoreType.DMA,
        ],
    )
    def _disp(e1_hbm, e2_hbm, gated_hbm, hist_hbm, gath_hbm, pos0_hbm,
              pos1_hbm, meta_hbm, e1_v, e2_v, grows_v, pos0_v, pos1_v,
              allh_v, meta_v, sem0, sem1):
        wid = lax.axis_index("s")
        base = wid * TPW
        lane = lax.iota(jnp.int32, 16)

        pltpu.sync_copy(e1_hbm.at[pl.ds(base, TPW)], e1_v)
        pltpu.sync_copy(e2_hbm.at[pl.ds(base, TPW)], e2_v)
        pltpu.sync_copy(gated_hbm.at[pl.ds(base, TPW)], grows_v)
        pltpu.sync_copy(hist_hbm, allh_v)

        total = jnp.zeros((16,), jnp.int32)
        pref = jnp.zeros((16,), jnp.int32)
        for t in range(NW):
            hv = allh_v[t]
            total = total + hv
            pref = pref + hv * (t < wid).astype(jnp.int32)
        valid = (lane < NE).astype(jnp.int32)
        total = total * valid
        padded = ((total + BLKG - 1) // BLKG) * BLKG
        cum = plsc.cumsum(padded)
        off = cum - padded
        run = off + pref

        for src, dst in ((e1_v, pos0_v), (e2_v, pos1_v)):
            for j in range(nv):
                v = src[pl.ds(j * 16, 16)]
                pos = jnp.zeros((16,), jnp.int32)
                for e in range(NE):
                    m = v == e
                    mi = m.astype(jnp.int32)
                    r = plsc.cumsum(mi) - 1
                    be = jnp.sum(jnp.where(lane == e, run, 0))
                    pos = jnp.where(m, be + r, pos)
                    cnt = jnp.sum(mi)
                    run = run + jnp.where(lane == e, cnt, 0)
                pos = jnp.minimum(jnp.maximum(pos, 0), P_MAX - 1)
                dst[j // 8, pl.ds((j % 8) * 16, 16)] = pos

        pltpu.sync_copy(pos0_v, pos0_hbm.at[pl.ds(wid * nr, nr)])
        pltpu.sync_copy(pos1_v, pos1_hbm.at[pl.ds(wid * nr, nr)])

        cps = []
        for r in range(nr):
            rows = grows_v.at[pl.ds(r * 128, 128)]
            cps.append(pltpu.async_copy(rows, gath_hbm.at[pos0_v.at[r]],
                                        sem0))
            cps.append(pltpu.async_copy(rows, gath_hbm.at[pos1_v.at[r]],
                                        sem1))
        for cp in cps:
            cp.wait()

        @pl.when(wid == 0)
        def _():
            nblocks = jnp.sum(jnp.where(lane == NE - 1, cum, 0)) // BLKG
            for j in range(META_N // 16):
                nbv = lane + j * 16
                start = nbv * BLKG
                be = jnp.zeros((16,), jnp.int32)
                for e in range(NE):
                    oe = jnp.sum(jnp.where(lane == e, off, 0))
                    pe = jnp.sum(jnp.where(lane == e, padded, 0))
                    inb = (start >= oe) & (start < oe + pe)
                    be = jnp.where(inb, e, be)
                if j == 3:
                    be = jnp.where(lane == 0, nblocks, be)
                meta_v[pl.ds(j * 16, 16)] = be
            pltpu.sync_copy(meta_v, meta_hbm)

    return _disp(e1, e2, gated, hist)


def _sc_dispatch_diag(e1, e2, run0):
    mesh = plsc.VectorSubcoreMesh(core_axis_name="c", subcore_axis_name="s",
                                  num_cores=1)
    nv = TPW // 16
    nr = TPW // 128

    @functools.partial(
        pl.kernel, mesh=mesh,
        compiler_params=pltpu.CompilerParams(needs_layout_passes=False),
        out_type=[
            jax.ShapeDtypeStruct((B // 128, 128), jnp.int32),
            jax.ShapeDtypeStruct((B // 128, 128), jnp.int32),
        ],
        scratch_types=[
            pltpu.VMEM((TPW,), jnp.int32),
            pltpu.VMEM((TPW,), jnp.int32),
            pltpu.VMEM((1, 16), jnp.int32),
            pltpu.VMEM((2, 128), jnp.int32),
            pltpu.VMEM((2, 128), jnp.int32),
        ],
    )
    def _disp(e1_hbm, e2_hbm, run_hbm, pos0_hbm, pos1_hbm,
              e1_v, e2_v, run_v, pos0_v, pos1_v):
        wid = lax.axis_index("s")
        base = wid * TPW
        lane = lax.iota(jnp.int32, 16)

        pltpu.sync_copy(e1_hbm.at[pl.ds(base, TPW)], e1_v)
        pltpu.sync_copy(e2_hbm.at[pl.ds(base, TPW)], e2_v)
        pltpu.sync_copy(run_hbm.at[pl.ds(wid, 1)], run_v)
        run = run_v[0]

        for src, dst in ((e1_v, pos0_v), (e2_v, pos1_v)):
            for j in range(nv):
                v = src[pl.ds(j * 16, 16)]
                pos = jnp.zeros((16,), jnp.int32)
                for e in range(NE):
                    m = v == e
                    mi = m.astype(jnp.int32)
                    r = plsc.cumsum(mi) - 1
                    be = jnp.sum(jnp.where(lane == e, run, 0))
                    pos = jnp.where(m, be + r, pos)
                    cnt = jnp.sum(mi)
                    run = run + jnp.where(lane == e, cnt, 0)
                pos = jnp.minimum(jnp.maximum(pos, 0), P_MAX - 1)
                dst[j // 8, pl.ds((j % 8) * 16, 16)] = pos

        pltpu.sync_copy(pos0_v, pos0_hbm.at[pl.ds(wid * nr, nr)])
        pltpu.sync_copy(pos1_v, pos1_hbm.at[pl.ds(wid * nr, nr)])

    return _disp(e1, e2, run0)


def _run0_emul(e1, e2):
    eall = jnp.concatenate([e1, e2]).reshape(2, NW, TPW)
    hist = jnp.sum(
        (eall[:, :, :, None] == jnp.arange(NE)[None, None, None, :])
        .astype(jnp.int32), axis=(0, 2))
    total = jnp.sum(hist, axis=0)
    padded = ((total + BLKG - 1) // BLKG) * BLKG
    off = jnp.cumsum(padded) - padded
    pref = jnp.cumsum(hist, axis=0) - hist
    run0 = off[None, :] + pref
    return jnp.pad(run0, ((0, 0), (0, 16 - NE)))


def _sc_gather2(eo, pos0, pos1):
    mesh = plsc.VectorSubcoreMesh(core_axis_name="c", subcore_axis_name="s",
                                  num_cores=1)
    nr = TPW // 128

    @functools.partial(
        pl.kernel, mesh=mesh,
        compiler_params=pltpu.CompilerParams(needs_layout_passes=False),
        out_type=[
            jax.ShapeDtypeStruct((B, EO), jnp.float32),
            jax.ShapeDtypeStruct((B, EO), jnp.float32),
        ],
        scratch_types=[
            pltpu.VMEM((2, 128), jnp.int32),
            pltpu.VMEM((2, 128), jnp.int32),
            pltpu.VMEM((TPW, EO), jnp.float32),
            pltpu.VMEM((TPW, EO), jnp.float32),
            pltpu.SemaphoreType.DMA,
            pltpu.SemaphoreType.DMA,
        ],
    )
    def _comb(eo_hbm, pos0_hbm, pos1_hbm, out0_hbm, out1_hbm,
              pos0_v, pos1_v, r0_v, r1_v, sem0, sem1):
        wid = lax.axis_index("s")
        base = wid * TPW
        pltpu.sync_copy(pos0_hbm.at[pl.ds(wid * nr, nr)], pos0_v)
        pltpu.sync_copy(pos1_hbm.at[pl.ds(wid * nr, nr)], pos1_v)
        cps = []
        for r in range(nr):
            cps.append(pltpu.async_copy(eo_hbm.at[pos0_v.at[r]],
                                        r0_v.at[pl.ds(r * 128, 128)], sem0))
            cps.append(pltpu.async_copy(eo_hbm.at[pos1_v.at[r]],
                                        r1_v.at[pl.ds(r * 128, 128)], sem1))
        for cp in cps:
            cp.wait()
        pltpu.sync_copy(r0_v, out0_hbm.at[pl.ds(base, TPW)])
        pltpu.sync_copy(r1_v, out1_hbm.at[pl.ds(base, TPW)])

    return _comb(eo, pos0, pos1)


def _forward(features, coin_id, regime_id, account, temporal, params,
             interpret=False, use_sc=True):
    w = _flatten_params(params)
    pre_names = tuple(n for n in w if _is_pre(n))
    post_names = tuple(n for n in w if _is_post(n))
    coin2 = coin_id.astype(jnp.int32).reshape(B, 1)
    reg2 = regime_id.astype(jnp.int32).reshape(B, 1)

    pre_w = [w[n] for n in pre_names]
    in_specs = [
        pl.BlockSpec((BLK, FEAT_DIM), lambda i: (i, 0)),
        pl.BlockSpec((BLK, 1), lambda i: (i, 0)),
        pl.BlockSpec((BLK, 1), lambda i: (i, 0)),
        pl.BlockSpec((BLK, N_ACC), lambda i: (i, 0)),
        pl.BlockSpec((BLK, N_TEMP), lambda i: (i, 0)),
    ] + [_const_spec(a) for a in pre_w]
    gated, ctx, eidx, ew = pl.pallas_call(
        functools.partial(_pre_body, pre_names),
        grid=(B // BLK,),
        in_specs=in_specs,
        out_specs=[
            pl.BlockSpec((BLK, EO), lambda i: (i, 0)),
            pl.BlockSpec((BLK, 64), lambda i: (i, 0)),
            pl.BlockSpec((BLK, NE), lambda i: (i, 0)),
            pl.BlockSpec((BLK, NE), lambda i: (i, 0)),
        ],
        out_shape=[
            jax.ShapeDtypeStruct((B, EO), jnp.float32),
            jax.ShapeDtypeStruct((B, 64), jnp.float32),
            jax.ShapeDtypeStruct((B, NE), jnp.int32),
            jax.ShapeDtypeStruct((B, NE), jnp.float32),
        ],
        interpret=interpret,
    )(features, coin2, reg2, account, temporal, *pre_w)

    e1 = eidx[:, 0]
    e2 = eidx[:, 1]
    w1col = ew[:, 0:1]
    w2col = ew[:, 1:2]
    if use_sc:
        gathered, pos0, pos1, meta = _sc_dispatch(gated, e1, e2)
        eo = _run_grouped(gathered, meta, w, interpret=interpret)
        m0, m1 = _sc_gather2(eo, pos0, pos1)
    else:
        gathered, pos0, pos1, meta = _dispatch_emul(gated, e1, e2)
        eo = _run_grouped(gathered, meta, w, interpret=interpret)
        m0, m1 = eo[pos0], eo[pos1]

    post_w = [w[n] for n in post_names]
    out = pl.pallas_call(
        functools.partial(_post_body, post_names),
        grid=(B // BLK,),
        in_specs=[
            pl.BlockSpec((BLK, EO), lambda i: (i, 0)),
            pl.BlockSpec((BLK, EO), lambda i: (i, 0)),
            pl.BlockSpec((BLK, 1), lambda i: (i, 0)),
            pl.BlockSpec((BLK, 1), lambda i: (i, 0)),
            pl.BlockSpec((BLK, 64), lambda i: (i, 0)),
        ] + [_const_spec(a) for a in post_w],
        out_specs=pl.BlockSpec((BLK, 98), lambda i: (i, 0)),
        out_shape=jax.ShapeDtypeStruct((B, 98), jnp.float32),
        interpret=interpret,
    )(m0, m1, w1col, w2col, ctx, *post_w)
    return out


def kernel(features, coin_id, regime_id, account, temporal, params):
    return _forward(features, coin_id, regime_id, account, temporal, params,
                    use_sc=True)


# speedup vs baseline: 1.0361x; 1.0361x over previous
import functools

import jax
import jax.numpy as jnp
from jax import lax
from jax.experimental import pallas as pl
from jax.experimental.pallas import tpu as pltpu
from jax.experimental.pallas import tpu_sc as plsc

B = 4096
BLK = 1024
BLKG = 256
P_MAX = 10240
NBLK_MAX = P_MAX // BLKG
META_N = 64
NW = 16
TPW = B // NW
N_COINS = 250
COIN_DIM = 32
REG_DIM = 16
N_ACC = 4
N_TEMP = 40
EH = 256
EO = 128
NE = 8
FEAT_DIM = 256
PART_NAMES = ("price", "volume", "orderflow", "derived")

_SQRT2 = 1.4142135623730951
_RSQRT_EO = 1.0 / (EO ** 0.5)


def _gelu(x):
    return 0.5 * x * (1.0 + lax.erf(x / _SQRT2))


def _ln(x, g, b, eps=1e-5):
    m = jnp.mean(x, axis=-1, keepdims=True)
    xc = x - m
    v = jnp.mean(xc * xc, axis=-1, keepdims=True)
    return xc * lax.rsqrt(v + eps) * g + b


def _dot(x, w):
    return jnp.dot(x, w, preferred_element_type=jnp.float32)


def _flatten_params(p):
    f32 = jnp.float32
    w = {}
    w["coin_emb"] = jnp.zeros((256, COIN_DIM), f32).at[:N_COINS].set(
        p["coin_emb"])
    w["regime_emb"] = jnp.zeros((8, REG_DIM), f32).at[:4].set(p["regime_emb"])
    w["temp1_w"] = p["temp1"]["w"]
    w["temp1_b"] = p["temp1"]["b"][None]
    w["temp2_w"] = p["temp2"]["w"]
    w["temp2_b"] = p["temp2"]["b"][None]
    w["temp_lng"] = p["temp_lng"][None]
    w["temp_lnb"] = p["temp_lnb"][None]
    for i, name in enumerate(PART_NAMES):
        ep = p["feat_experts"][name]
        w[f"fe{i}_w1"] = ep["w1"]
        w[f"fe{i}_b1"] = ep["b1"][None]
        w[f"fe{i}_w2"] = ep["w2"]
        w[f"fe{i}_b2"] = ep["b2"][None]
        w[f"fe{i}_w3"] = ep["w3"]
        w[f"fe{i}_b3"] = ep["b3"][None]
        w[f"fe{i}_wr"] = ep["wr"]
        w[f"fe{i}_br"] = ep["br"][None]
        w[f"fe{i}_lng"] = ep["lng"][None]
        w[f"fe{i}_lnb"] = ep["lnb"][None]
        w[f"gk{i}_w"] = p["gate_keys"][name]["w"]
        w[f"gk{i}_b"] = p["gate_keys"][name]["b"][None]
    cw = p["context"]["w"]
    w["ctx_wa"] = cw[0:N_ACC]
    w["ctx_wc"] = cw[N_ACC:N_ACC + COIN_DIM]
    w["ctx_wr"] = cw[N_ACC + COIN_DIM:N_ACC + COIN_DIM + REG_DIM]
    w["ctx_wt"] = cw[N_ACC + COIN_DIM + REG_DIM:]
    w["ctx_b"] = p["context"]["b"][None]
    qw = p["gate_q"]["w"]
    w["gq_cat"] = qw[:4 * EO]
    w["gq_ctx"] = qw[4 * EO:]
    w["gq_b"] = p["gate_q"]["b"][None]
    rw = p["router1"]["w"]
    w["r1_g"] = rw[:EO]
    w["r1_r"] = rw[EO:]
    w["r1_b"] = p["router1"]["b"][None]
    w["r2_w"] = p["router2"]["w"]
    w["r2_b"] = p["router2"]["b"][None]
    for e in range(NE):
        ep = p["moe_experts"][e]
        w[f"moe{e}_w1"] = ep["w1"]
        w[f"moe{e}_b1"] = ep["b1"][None]
        w[f"moe{e}_w2"] = ep["w2"]
        w[f"moe{e}_b2"] = ep["b2"][None]
        w[f"moe{e}_w3"] = ep["w3"]
        w[f"moe{e}_b3"] = ep["b3"][None]
        w[f"moe{e}_lng"] = ep["lng"][None]
        w[f"moe{e}_lnb"] = ep["lnb"][None]
    fw = p["fus1"]["w"]
    w["f1_m"] = fw[:EO]
    w["f1_c"] = fw[EO:]
    w["f1_b"] = p["fus1"]["b"][None]
    w["f_ln1g"] = p["fus_ln1g"][None]
    w["f_ln1b"] = p["fus_ln1b"][None]
    w["f2_w"] = p["fus2"]["w"]
    w["f2_b"] = p["fus2"]["b"][None]
    w["f_ln2g"] = p["fus_ln2g"][None]
    w["f_ln2b"] = p["fus_ln2b"][None]
    for g in range(4):
        hp = p["heads"][g]
        for hname in ("lab", "mae", "mfe"):
            for lyr in ("1", "2"):
                w[f"hd_{hname}{lyr}_{g}_w"] = hp[hname + lyr]["w"]
                w[f"hd_{hname}{lyr}_{g}_b"] = hp[hname + lyr]["b"][None]
    for nm in ("conf1", "conf2", "lev1", "lev2"):
        w[nm + "_w"] = p[nm]["w"]
        w[nm + "_b"] = p[nm]["b"][None]
    return w


def _body(names, *refs):
    feats_ref, coin_ref, reg_ref, acct_ref, temp_ref = refs[:5]
    out_ref = refs[-1]
    w = {n: r for n, r in zip(names, refs[5:-1])}

    feats = feats_ref[...]
    coin_id = coin_ref[...]
    regime_id = reg_ref[...]
    acct = acct_ref[...]
    temporal = temp_ref[...]

    iota_c = lax.broadcasted_iota(jnp.int32, (BLK, 256), 1)
    oh_c = (iota_c == coin_id).astype(jnp.float32)
    coin_emb = _dot(oh_c, w["coin_emb"][...])
    iota_r = lax.broadcasted_iota(jnp.int32, (BLK, 8), 1)
    oh_r = (iota_r == regime_id).astype(jnp.float32)
    regime_emb = _dot(oh_r, w["regime_emb"][...])

    t = _gelu(_dot(temporal, w["temp1_w"][...]) + w["temp1_b"][...])
    t = _dot(t, w["temp2_w"][...]) + w["temp2_b"][...]
    temporal_enc = _ln(t, w["temp_lng"][...], w["temp_lnb"][...])

    feat_outs = []
    for i in range(4):
        x = feats[:, i * 64:(i + 1) * 64]
        h = _gelu(_dot(x, w[f"fe{i}_w1"][...]) + w[f"fe{i}_b1"][...])
        h = _gelu(_dot(h, w[f"fe{i}_w2"][...]) + w[f"fe{i}_b2"][...])
        h = _dot(h, w[f"fe{i}_w3"][...]) + w[f"fe{i}_b3"][...]
        res = _dot(x, w[f"fe{i}_wr"][...]) + w[f"fe{i}_br"][...]
        feat_outs.append(_ln(h + res, w[f"fe{i}_lng"][...],
                             w[f"fe{i}_lnb"][...]))

    ctx = (_dot(acct, w["ctx_wa"][...]) + _dot(coin_emb, w["ctx_wc"][...])
           + _dot(regime_emb, w["ctx_wr"][...])
           + _dot(temporal_enc, w["ctx_wt"][...]) + w["ctx_b"][...])
    context_enc = _gelu(ctx)

    fcat = jnp.concatenate(feat_outs, axis=-1)
    q = (w["gq_b"][...] + _dot(context_enc, w["gq_ctx"][...])
         + _dot(fcat, w["gq_cat"][...]))
    scores = []
    for i in range(4):
        k = _dot(feat_outs[i], w[f"gk{i}_w"][...]) + w[f"gk{i}_b"][...]
        scores.append(jnp.sum(q * k, axis=-1, keepdims=True) * _RSQRT_EO)
    smax = jnp.maximum(jnp.maximum(scores[0], scores[1]),
                       jnp.maximum(scores[2], scores[3]))
    exps = [jnp.exp(s - smax) for s in scores]
    denom = exps[0] + exps[1] + exps[2] + exps[3]
    gated = jnp.zeros((BLK, EO), jnp.float32)
    for i in range(4):
        gated = gated + (exps[i] / denom) * feat_outs[i]

    rh = _gelu(_dot(gated, w["r1_g"][...]) + _dot(regime_emb, w["r1_r"][...])
               + w["r1_b"][...])
    logits = _dot(rh, w["r2_w"][...]) + w["r2_b"][...]
    iota8 = lax.broadcasted_iota(jnp.int32, (BLK, NE), 1)
    m1 = jnp.max(logits, axis=-1, keepdims=True)
    i1 = jnp.min(jnp.where(logits == m1, iota8, NE), axis=-1, keepdims=True)
    masked = jnp.where(iota8 == i1, -1e30, logits)
    m2 = jnp.max(masked, axis=-1, keepdims=True)
    i2 = jnp.min(jnp.where(masked == m2, iota8, NE), axis=-1, keepdims=True)
    e2 = jnp.exp(m2 - m1)
    w1c = 1.0 / (1.0 + e2)
    w2c = e2 * w1c
    coefs = (jnp.where(iota8 == i1, w1c, 0.0)
             + jnp.where(iota8 == i2, w2c, 0.0))

    moe = jnp.zeros((BLK, EO), jnp.float32)
    for e in range(NE):
        h = _gelu(_dot(gated, w[f"moe{e}_w1"][...]) + w[f"moe{e}_b1"][...])
        h = _gelu(_dot(h, w[f"moe{e}_w2"][...]) + w[f"moe{e}_b2"][...])
        h = _dot(h, w[f"moe{e}_w3"][...]) + w[f"moe{e}_b3"][...]
        eo = _ln(h + gated, w[f"moe{e}_lng"][...], w[f"moe{e}_lnb"][...])
        moe = moe + lax.slice_in_dim(coefs, e, e + 1, axis=1) * eo

    f = _gelu(_dot(moe, w["f1_m"][...]) + _dot(context_enc, w["f1_c"][...])
              + w["f1_b"][...])
    f = _ln(f, w["f_ln1g"][...], w["f_ln1b"][...])
    f = _gelu(_dot(f, w["f2_w"][...]) + w["f2_b"][...])
    f = _ln(f, w["f_ln2g"][...], w["f_ln2b"][...])

    pieces = []
    for hname in ("lab", "mae", "mfe"):
        for g in range(4):
            h1 = _gelu(_dot(f, w[f"hd_{hname}1_{g}_w"][...])
                       + w[f"hd_{hname}1_{g}_b"][...])
            pieces.append(_dot(h1, w[f"hd_{hname}2_{g}_w"][...])
                          + w[f"hd_{hname}2_{g}_b"][...])
    c = _gelu(_dot(f, w["conf1_w"][...]) + w["conf1_b"][...])
    pieces.append(jax.nn.sigmoid(_dot(c, w["conf2_w"][...])
                                 + w["conf2_b"][...]))
    lv = _gelu(_dot(f, w["lev1_w"][...]) + w["lev1_b"][...])
    pieces.append(jax.nn.sigmoid(_dot(lv, w["lev2_w"][...])
                                 + w["lev2_b"][...]))
    out_ref[...] = jnp.concatenate(pieces, axis=-1)


def _const_spec(arr):
    nd = arr.ndim
    return pl.BlockSpec(arr.shape, lambda i, _nd=nd: (0,) * _nd)


def _is_post(n):
    return (n.startswith(("f1_", "f2_", "f_ln", "hd_"))
            or n.startswith(("conf", "lev")))


def _is_pre(n):
    return not _is_post(n) and not n.startswith("moe")


def _pre_body(names, *refs):
    feats_ref, coin_ref, reg_ref, acct_ref, temp_ref = refs[:5]
    gated_ref, ctx_ref, eidx_ref, ew_ref, hist_ref = refs[-5:]
    w = {n: r for n, r in zip(names, refs[5:-5])}

    feats = feats_ref[...]
    coin_id = coin_ref[...]
    regime_id = reg_ref[...]
    acct = acct_ref[...]
    temporal = temp_ref[...]

    iota_c = lax.broadcasted_iota(jnp.int32, (BLK, 256), 1)
    oh_c = (iota_c == coin_id).astype(jnp.float32)
    coin_emb = _dot(oh_c, w["coin_emb"][...])
    iota_r = lax.broadcasted_iota(jnp.int32, (BLK, 8), 1)
    oh_r = (iota_r == regime_id).astype(jnp.float32)
    regime_emb = _dot(oh_r, w["regime_emb"][...])

    t = _gelu(_dot(temporal, w["temp1_w"][...]) + w["temp1_b"][...])
    t = _dot(t, w["temp2_w"][...]) + w["temp2_b"][...]
    temporal_enc = _ln(t, w["temp_lng"][...], w["temp_lnb"][...])

    feat_outs = []
    for i in range(4):
        x = feats[:, i * 64:(i + 1) * 64]
        h = _gelu(_dot(x, w[f"fe{i}_w1"][...]) + w[f"fe{i}_b1"][...])
        h = _gelu(_dot(h, w[f"fe{i}_w2"][...]) + w[f"fe{i}_b2"][...])
        h = _dot(h, w[f"fe{i}_w3"][...]) + w[f"fe{i}_b3"][...]
        res = _dot(x, w[f"fe{i}_wr"][...]) + w[f"fe{i}_br"][...]
        feat_outs.append(_ln(h + res, w[f"fe{i}_lng"][...],
                             w[f"fe{i}_lnb"][...]))

    ctx = (_dot(acct, w["ctx_wa"][...]) + _dot(coin_emb, w["ctx_wc"][...])
           + _dot(regime_emb, w["ctx_wr"][...])
           + _dot(temporal_enc, w["ctx_wt"][...]) + w["ctx_b"][...])
    context_enc = _gelu(ctx)

    fcat = jnp.concatenate(feat_outs, axis=-1)
    q = (w["gq_b"][...] + _dot(context_enc, w["gq_ctx"][...])
         + _dot(fcat, w["gq_cat"][...]))
    scores = []
    for i in range(4):
        k = _dot(feat_outs[i], w[f"gk{i}_w"][...]) + w[f"gk{i}_b"][...]
        scores.append(jnp.sum(q * k, axis=-1, keepdims=True) * _RSQRT_EO)
    smax = jnp.maximum(jnp.maximum(scores[0], scores[1]),
                       jnp.maximum(scores[2], scores[3]))
    exps = [jnp.exp(s - smax) for s in scores]
    denom = exps[0] + exps[1] + exps[2] + exps[3]
    gated = jnp.zeros((BLK, EO), jnp.float32)
    for i in range(4):
        gated = gated + (exps[i] / denom) * feat_outs[i]

    rh = _gelu(_dot(gated, w["r1_g"][...]) + _dot(regime_emb, w["r1_r"][...])
               + w["r1_b"][...])
    logits = _dot(rh, w["r2_w"][...]) + w["r2_b"][...]
    iota8 = lax.broadcasted_iota(jnp.int32, (BLK, NE), 1)
    m1 = jnp.max(logits, axis=-1, keepdims=True)
    i1 = jnp.min(jnp.where(logits == m1, iota8, NE), axis=-1, keepdims=True)
    masked = jnp.where(iota8 == i1, -1e30, logits)
    m2 = jnp.max(masked, axis=-1, keepdims=True)
    i2 = jnp.min(jnp.where(masked == m2, iota8, NE), axis=-1, keepdims=True)
    e2 = jnp.exp(m2 - m1)
    w1c = 1.0 / (1.0 + e2)
    w2c = e2 * w1c

    gated_ref[...] = gated
    ctx_ref[...] = context_enc
    eidx_ref[...] = jnp.where(iota8 == 0, i1, jnp.where(iota8 == 1, i2, 0))
    ew_ref[...] = jnp.where(iota8 == 0, w1c, jnp.where(iota8 == 1, w2c, 0.0))

    cnt = ((iota8 == i1).astype(jnp.int32) + (iota8 == i2).astype(jnp.int32))
    rows = [jnp.sum(cnt[k * TPW:(k + 1) * TPW], axis=0, keepdims=True)
            for k in range(BLK // TPW)]
    h8 = jnp.concatenate(rows, axis=0)
    h16 = jnp.concatenate(
        [h8, jnp.zeros((BLK // TPW, 16 - NE), jnp.int32)], axis=1)
    hist_ref[pl.ds(pl.program_id(0) * (BLK // TPW), BLK // TPW), :] = h16


def _post_body(names, *refs):
    m0_ref, m1_ref, w1_ref, w2_ref, ctx_ref = refs[:5]
    out_ref = refs[-1]
    w = {n: r for n, r in zip(names, refs[5:-1])}
    moe = w1_ref[...] * m0_ref[...] + w2_ref[...] * m1_ref[...]
    context_enc = ctx_ref[...]
    f = _gelu(_dot(moe, w["f1_m"][...]) + _dot(context_enc, w["f1_c"][...])
              + w["f1_b"][...])
    f = _ln(f, w["f_ln1g"][...], w["f_ln1b"][...])
    f = _gelu(_dot(f, w["f2_w"][...]) + w["f2_b"][...])
    f = _ln(f, w["f_ln2g"][...], w["f_ln2b"][...])
    pieces = []
    for hname in ("lab", "mae", "mfe"):
        for g in range(4):
            h1 = _gelu(_dot(f, w[f"hd_{hname}1_{g}_w"][...])
                       + w[f"hd_{hname}1_{g}_b"][...])
            pieces.append(_dot(h1, w[f"hd_{hname}2_{g}_w"][...])
                          + w[f"hd_{hname}2_{g}_b"][...])
    c = _gelu(_dot(f, w["conf1_w"][...]) + w["conf1_b"][...])
    pieces.append(jax.nn.sigmoid(_dot(c, w["conf2_w"][...])
                                 + w["conf2_b"][...]))
    lv = _gelu(_dot(f, w["lev1_w"][...]) + w["lev1_b"][...])
    pieces.append(jax.nn.sigmoid(_dot(lv, w["lev2_w"][...])
                                 + w["lev2_b"][...]))
    out_ref[...] = jnp.concatenate(pieces, axis=-1)


def _grouped_body(m_ref, x_ref, w1_ref, w2_ref, w3_ref, b1_ref, b2_ref,
                  b3_ref, lng_ref, lnb_ref, o_ref):
    i = pl.program_id(0)

    @pl.when(i < m_ref[48])
    def _():
        x = x_ref[...]
        h = _gelu(_dot(x, w1_ref[0]) + b1_ref[0])
        h = _gelu(_dot(h, w2_ref[0]) + b2_ref[0])
        h = _dot(h, w3_ref[0]) + b3_ref[0]
        o_ref[...] = _ln(h + x, lng_ref[0], lnb_ref[0])


def _run_grouped(gathered, meta, w, interpret=False):
    grid_spec = pltpu.PrefetchScalarGridSpec(
        num_scalar_prefetch=1,
        grid=(NBLK_MAX,),
        in_specs=[
            pl.BlockSpec((BLKG, EO), lambda i, m: (i, 0)),
            pl.BlockSpec((1, EO, EH), lambda i, m: (m[i], 0, 0)),
            pl.BlockSpec((1, EH, EH), lambda i, m: (m[i], 0, 0)),
            pl.BlockSpec((1, EH, EO), lambda i, m: (m[i], 0, 0)),
            pl.BlockSpec((1, 1, EH), lambda i, m: (m[i], 0, 0)),
            pl.BlockSpec((1, 1, EH), lambda i, m: (m[i], 0, 0)),
            pl.BlockSpec((1, 1, EO), lambda i, m: (m[i], 0, 0)),
            pl.BlockSpec((1, 1, EO), lambda i, m: (m[i], 0, 0)),
            pl.BlockSpec((1, 1, EO), lambda i, m: (m[i], 0, 0)),
        ],
        out_specs=pl.BlockSpec((BLKG, EO), lambda i, m: (i, 0)),
    )
    mw1 = jnp.stack([w[f"moe{e}_w1"] for e in range(NE)])
    mw2 = jnp.stack([w[f"moe{e}_w2"] for e in range(NE)])
    mw3 = jnp.stack([w[f"moe{e}_w3"] for e in range(NE)])
    mb1 = jnp.stack([w[f"moe{e}_b1"] for e in range(NE)])
    mb2 = jnp.stack([w[f"moe{e}_b2"] for e in range(NE)])
    mb3 = jnp.stack([w[f"moe{e}_b3"] for e in range(NE)])
    mlg = jnp.stack([w[f"moe{e}_lng"] for e in range(NE)])
    mlb = jnp.stack([w[f"moe{e}_lnb"] for e in range(NE)])
    return pl.pallas_call(
        _grouped_body,
        grid_spec=grid_spec,
        out_shape=jax.ShapeDtypeStruct((P_MAX, EO), jnp.float32),
        interpret=interpret,
    )(meta, gathered, mw1, mw2, mw3, mb1, mb2, mb3, mlg, mlb)


def _sc_dispatch(gated, e1, e2, hist):
    mesh = plsc.VectorSubcoreMesh(core_axis_name="c", subcore_axis_name="s",
                                  num_cores=1)
    nv = TPW // 16
    nr = TPW // 128

    @functools.partial(
        pl.kernel, mesh=mesh,
        compiler_params=pltpu.CompilerParams(needs_layout_passes=False),
        out_type=[
            jax.ShapeDtypeStruct((P_MAX, EO), jnp.float32),
            jax.ShapeDtypeStruct((B // 128, 128), jnp.int32),
            jax.ShapeDtypeStruct((B // 128, 128), jnp.int32),
            jax.ShapeDtypeStruct((META_N,), jnp.int32),
        ],
        scratch_types=[
            pltpu.VMEM((TPW,), jnp.int32),
            pltpu.VMEM((TPW,), jnp.int32),
            pltpu.VMEM((TPW, EO), jnp.float32),
            pltpu.VMEM((2, 128), jnp.int32),
            pltpu.VMEM((2, 128), jnp.int32),
            pltpu.VMEM((NW, 16), jnp.int32),
            pltpu.VMEM((META_N,), jnp.int32),
            pltpu.SemaphoreType.DMA,
            pltpu.SemaphoreType.DMA,
        ],
    )
    def _disp(e1_hbm, e2_hbm, gated_hbm, hist_hbm, gath_hbm, pos0_hbm,
              pos1_hbm, meta_hbm, e1_v, e2_v, grows_v, pos0_v, pos1_v,
              allh_v, meta_v, sem0, sem1):
        wid = lax.axis_index("s")
        base = wid * TPW
        lane = lax.iota(jnp.int32, 16)

        pltpu.sync_copy(e1_hbm.at[pl.ds(base, TPW)], e1_v)
        pltpu.sync_copy(e2_hbm.at[pl.ds(base, TPW)], e2_v)
        pltpu.sync_copy(gated_hbm.at[pl.ds(base, TPW)], grows_v)
        pltpu.sync_copy(hist_hbm, allh_v)

        total = jnp.zeros((16,), jnp.int32)
        pref = jnp.zeros((16,), jnp.int32)
        for t in range(NW):
            hv = allh_v[t]
            total = total + hv
            pref = pref + hv * (t < wid).astype(jnp.int32)
        valid = (lane < NE).astype(jnp.int32)
        total = total * valid
        padded = ((total + BLKG - 1) // BLKG) * BLKG
        cum = plsc.cumsum(padded)
        off = cum - padded
        run = off + pref

        for src, dst in ((e1_v, pos0_v), (e2_v, pos1_v)):
            for j in range(nv):
                v = src[pl.ds(j * 16, 16)]
                pos = jnp.zeros((16,), jnp.int32)
                for e in range(NE):
                    m = v == e
                    mi = m.astype(jnp.int32)
                    r = plsc.cumsum(mi) - 1
                    be = jnp.sum(jnp.where(lane == e, run, 0))
                    pos = jnp.where(m, be + r, pos)
                    cnt = jnp.sum(mi)
                    run = run + jnp.where(lane == e, cnt, 0)
                pos = jnp.minimum(jnp.maximum(pos, 0), P_MAX - 1)
                dst[j // 8, pl.ds((j % 8) * 16, 16)] = pos

        pltpu.sync_copy(pos0_v, pos0_hbm.at[pl.ds(wid * nr, nr)])
        pltpu.sync_copy(pos1_v, pos1_hbm.at[pl.ds(wid * nr, nr)])

        cps = []
        for r in range(nr):
            rows = grows_v.at[pl.ds(r * 128, 128)]
            cps.append(pltpu.async_copy(rows, gath_hbm.at[pos0_v.at[r]],
                                        sem0))
            cps.append(pltpu.async_copy(rows, gath_hbm.at[pos1_v.at[r]],
                                        sem1))
        for cp in cps:
            cp.wait()

        @pl.when(wid == 0)
        def _():
            nblocks = jnp.sum(jnp.where(lane == NE - 1, cum, 0)) // BLKG
            for j in range(META_N // 16):
                nbv = lane + j * 16
                start = nbv * BLKG
                be = jnp.zeros((16,), jnp.int32)
                for e in range(NE):
                    oe = jnp.sum(jnp.where(lane == e, off, 0))
                    pe = jnp.sum(jnp.where(lane == e, padded, 0))
                    inb = (start >= oe) & (start < oe + pe)
                    be = jnp.where(inb, e, be)
                if j == 3:
                    be = jnp.where(lane == 0, nblocks, be)
                meta_v[pl.ds(j * 16, 16)] = be
            pltpu.sync_copy(meta_v, meta_hbm)

    return _disp(e1, e2, gated, hist)


def _sc_gather2(eo, pos0, pos1):
    mesh = plsc.VectorSubcoreMesh(core_axis_name="c", subcore_axis_name="s",
                                  num_cores=1)
    nr = TPW // 128

    @functools.partial(
        pl.kernel, mesh=mesh,
        compiler_params=pltpu.CompilerParams(needs_layout_passes=False),
        out_type=[
            jax.ShapeDtypeStruct((B, EO), jnp.float32),
            jax.ShapeDtypeStruct((B, EO), jnp.float32),
        ],
        scratch_types=[
            pltpu.VMEM((2, 128), jnp.int32),
            pltpu.VMEM((2, 128), jnp.int32),
            pltpu.VMEM((TPW, EO), jnp.float32),
            pltpu.VMEM((TPW, EO), jnp.float32),
            pltpu.SemaphoreType.DMA,
            pltpu.SemaphoreType.DMA,
        ],
    )
    def _comb(eo_hbm, pos0_hbm, pos1_hbm, out0_hbm, out1_hbm,
              pos0_v, pos1_v, r0_v, r1_v, sem0, sem1):
        wid = lax.axis_index("s")
        base = wid * TPW
        pltpu.sync_copy(pos0_hbm.at[pl.ds(wid * nr, nr)], pos0_v)
        pltpu.sync_copy(pos1_hbm.at[pl.ds(wid * nr, nr)], pos1_v)
        cps = []
        for r in range(nr):
            cps.append(pltpu.async_copy(eo_hbm.at[pos0_v.at[r]],
                                        r0_v.at[pl.ds(r * 128, 128)], sem0))
            cps.append(pltpu.async_copy(eo_hbm.at[pos1_v.at[r]],
                                        r1_v.at[pl.ds(r * 128, 128)], sem1))
        for cp in cps:
            cp.wait()
        pltpu.sync_copy(r0_v, out0_hbm.at[pl.ds(base, TPW)])
        pltpu.sync_copy(r1_v, out1_hbm.at[pl.ds(base, TPW)])

    return _comb(eo, pos0, pos1)


def _forward(features, coin_id, regime_id, account, temporal, params,
             interpret=False):
    w = _flatten_params(params)
    pre_names = tuple(n for n in w if _is_pre(n))
    post_names = tuple(n for n in w if _is_post(n))
    coin2 = coin_id.astype(jnp.int32).reshape(B, 1)
    reg2 = regime_id.astype(jnp.int32).reshape(B, 1)

    pre_w = [w[n] for n in pre_names]
    in_specs = [
        pl.BlockSpec((BLK, FEAT_DIM), lambda i: (i, 0)),
        pl.BlockSpec((BLK, 1), lambda i: (i, 0)),
        pl.BlockSpec((BLK, 1), lambda i: (i, 0)),
        pl.BlockSpec((BLK, N_ACC), lambda i: (i, 0)),
        pl.BlockSpec((BLK, N_TEMP), lambda i: (i, 0)),
    ] + [_const_spec(a) for a in pre_w]
    gated, ctx, eidx, ew, hist = pl.pallas_call(
        functools.partial(_pre_body, pre_names),
        grid=(B // BLK,),
        in_specs=in_specs,
        out_specs=[
            pl.BlockSpec((BLK, EO), lambda i: (i, 0)),
            pl.BlockSpec((BLK, 64), lambda i: (i, 0)),
            pl.BlockSpec((BLK, NE), lambda i: (i, 0)),
            pl.BlockSpec((BLK, NE), lambda i: (i, 0)),
            pl.BlockSpec((NW, 16), lambda i: (0, 0)),
        ],
        out_shape=[
            jax.ShapeDtypeStruct((B, EO), jnp.float32),
            jax.ShapeDtypeStruct((B, 64), jnp.float32),
            jax.ShapeDtypeStruct((B, NE), jnp.int32),
            jax.ShapeDtypeStruct((B, NE), jnp.float32),
            jax.ShapeDtypeStruct((NW, 16), jnp.int32),
        ],
        interpret=interpret,
    )(features, coin2, reg2, account, temporal, *pre_w)

    e1 = eidx[:, 0]
    e2 = eidx[:, 1]
    w1col = ew[:, 0:1]
    w2col = ew[:, 1:2]
    gathered, pos0, pos1, meta = _sc_dispatch(gated, e1, e2, hist)
    eo = _run_grouped(gathered, meta, w, interpret=interpret)
    m0, m1 = _sc_gather2(eo, pos0, pos1)

    post_w = [w[n] for n in post_names]
    out = pl.pallas_call(
        functools.partial(_post_body, post_names),
        grid=(B // BLK,),
        in_specs=[
            pl.BlockSpec((BLK, EO), lambda i: (i, 0)),
            pl.BlockSpec((BLK, EO), lambda i: (i, 0)),
            pl.BlockSpec((BLK, 1), lambda i: (i, 0)),
            pl.BlockSpec((BLK, 1), lambda i: (i, 0)),
            pl.BlockSpec((BLK, 64), lambda i: (i, 0)),
        ] + [_const_spec(a) for a in post_w],
        out_specs=pl.BlockSpec((BLK, 98), lambda i: (i, 0)),
        out_shape=jax.ShapeDtypeStruct((B, 98), jnp.float32),
        interpret=interpret,
    )(m0, m1, w1col, w2col, ctx, *post_w)
    return out


def kernel(features, coin_id, regime_id, account, temporal, params):
    return _forward(features, coin_id, regime_id, account, temporal, params)
